# baseline (speedup 1.0000x reference)
import functools

import jax
import jax.numpy as jnp
from jax import lax
from jax.experimental import pallas as pl
from jax.experimental.pallas import tpu as pltpu
from jax.experimental.pallas import tpu_sc as plsc

_G = 5
_KORD = 3
_NLAYERS = 3
_NGRAPHS = 64
_NB = _G + _KORD
_H = 2.0 / _G
_G0 = -_KORD * _H - 1.0

_NC, _NS = 2, 16
_NW = _NC * _NS


def _make_sc_scatter_add(n_acc, n_edges, d, ch):
  epw = n_edges // _NW
  assert epw * _NW == n_edges and epw % ch == 0
  nch = epw // ch
  rps = n_acc // _NS
  assert rps * _NS == n_acc and rps % 8 == 0
  mesh = plsc.VectorSubcoreMesh(
      core_axis_name="c", subcore_axis_name="s", num_cores=_NC,
      num_subcores=_NS)

  @functools.partial(
      pl.kernel,
      mesh=mesh,
      out_type=jax.ShapeDtypeStruct((_NC, n_acc, d), jnp.float32),
      scratch_types=[
          pltpu.VMEM((ch,), jnp.int32),
          pltpu.VMEM((ch,), jnp.int32),
          pltpu.VMEM((ch, d), jnp.float32),
          pltpu.VMEM_SHARED((n_acc, d), jnp.float32),
          pltpu.SemaphoreType.DMA,
      ],
  )
  def sc_kernel(x_hbm, src_hbm, dst_hbm, zeros_hbm, out_hbm,
                src_v, dst_v, rows_v, acc_sh, gsem):
    c = lax.axis_index("c")
    s = lax.axis_index("s")
    wid = c * _NS + s
    pltpu.sync_copy(zeros_hbm.at[pl.ds(s * rps, rps)],
                    acc_sh.at[pl.ds(s * rps, rps)])
    plsc.subcore_barrier()
    base = wid * epw

    def step(t, carry):
      off = pl.multiple_of(base + t * ch, 8)
      pltpu.sync_copy(src_hbm.at[pl.ds(off, ch)], src_v)
      pltpu.sync_copy(dst_hbm.at[pl.ds(off, ch)], dst_v)
      pltpu.async_copy(x_hbm.at[src_v], rows_v, gsem).wait()
      pltpu.sync_copy(rows_v, acc_sh.at[dst_v], add=True)
      return carry

    lax.fori_loop(0, nch, step, 0)
    plsc.subcore_barrier()
    pltpu.sync_copy(acc_sh.at[pl.ds(s * rps, rps)],
                    out_hbm.at[c, pl.ds(s * rps, rps)])

  return sc_kernel


def _kan_block(x, bw, sw):
  sig = 1.0 / (1.0 + jnp.exp(-x))
  out = jnp.dot(x * sig, bw, preferred_element_type=jnp.float32)
  u = (x - _G0) * (1.0 / _H)
  cell = jnp.floor(u)
  t = u - cell
  t2 = t * t
  t3 = t2 * t
  n0 = t3 * (1.0 / 6.0)
  n1 = (1.0 + 3.0 * (t + t2 - t3)) * (1.0 / 6.0)
  n2 = (4.0 - 6.0 * t2 + 3.0 * t3) * (1.0 / 6.0)
  one_t = 1.0 - t
  n3 = one_t * one_t * one_t * (1.0 / 6.0)
  cols = []
  for j in range(_NB):
    m = cell - float(j)
    bj = jnp.where(m == 0.0, n0, 0.0)
    bj = jnp.where(m == 1.0, n1, bj)
    bj = jnp.where(m == 2.0, n2, bj)
    bj = jnp.where(m == 3.0, n3, bj)
    cols.append(bj)
  bcat = jnp.concatenate(cols, axis=1)
  out = out + jnp.dot(bcat, sw, preferred_element_type=jnp.float32)
  return out


def _kan_ref_dbg(x, p, k=_KORD):
  h = 2.0 / _G
  g = jnp.arange(-k, _G + k + 1, dtype=jnp.float32) * h - 1.0
  grid = jnp.broadcast_to(g, (x.shape[1], g.shape[0]))
  base = jax.nn.silu(x) @ p['base_w'].T
  xe = x[..., None]
  bases = ((xe >= grid[:, :-1]) & (xe < grid[:, 1:])).astype(x.dtype)
  for dd in range(1, k + 1):
    left = (xe - grid[:, : -(dd + 1)]) / (grid[:, dd:-1] - grid[:, : -(dd + 1)]) * bases[..., :-1]
    right = (grid[:, dd + 1:] - xe) / (grid[:, dd + 1:] - grid[:, 1:-dd]) * bases[..., 1:]
    bases = left + right
  sw = p['spline_w'] * p['spline_s'][..., None]
  spline = bases.reshape(bases.shape[0], -1) @ sw.reshape(sw.shape[0], -1).T
  return base + spline


def _kan_block2(x, bw, sw):
  out = jnp.dot(jax.nn.silu(x), bw, preferred_element_type=jnp.float32)
  u = (x - _G0) * (1.0 / _H)
  cell = jnp.floor(u)
  t = u - cell
  t2 = t * t
  t3 = t2 * t
  n0 = t3 * (1.0 / 6.0)
  n1 = (1.0 + 3.0 * (t + t2 - t3)) * (1.0 / 6.0)
  n2 = (4.0 - 6.0 * t2 + 3.0 * t3) * (1.0 / 6.0)
  one_t = 1.0 - t
  n3 = one_t * one_t * one_t * (1.0 / 6.0)
  cols = []
  for j in range(_NB):
    m = cell - float(j)
    bj = jnp.where(m == 0.0, n0, 0.0)
    bj = jnp.where(m == 1.0, n1, bj)
    bj = jnp.where(m == 2.0, n2, bj)
    bj = jnp.where(m == 3.0, n3, bj)
    cols.append(bj)
  bcat = jnp.stack(cols, axis=2).reshape(x.shape[0], -1)
  out = out + jnp.dot(bcat, sw, preferred_element_type=jnp.float32)
  return out


def _prep_kan_layer_fm(p, dpad=None):
  bw = p['base_w']
  sw = p['spline_w'] * p['spline_s'][..., None]
  fout, fin = bw.shape
  bw_t = bw.T
  sw_t = jnp.transpose(sw, (1, 2, 0)).reshape(fin * _NB, fout)
  if dpad is not None and fout < dpad:
    bw_t = jnp.pad(bw_t, ((0, 0), (0, dpad - fout)))
    sw_t = jnp.pad(sw_t, ((0, 0), (0, dpad - fout)))
  return bw_t, sw_t


def _kan_pair_body(x_ref, agg_ref, bw0_ref, sw0_ref, bw1_ref, sw1_ref,
                   h_ref, s1_ref, s2_ref):
  xin = x_ref[...] + agg_ref[0] + agg_ref[1]
  h1 = _kan_block(xin, bw0_ref[...], sw0_ref[...])
  h2 = _kan_block(h1, bw1_ref[...], sw1_ref[...])
  h_ref[...] = h2
  ps = jnp.sum(h2, axis=0, keepdims=True)
  pq = jnp.sum(h2 * h2, axis=0, keepdims=True)

  @pl.when(pl.program_id(0) == 0)
  def _init():
    s1_ref[...] = ps
    s2_ref[...] = pq

  @pl.when(pl.program_id(0) > 0)
  def _acc():
    s1_ref[...] += ps
    s2_ref[...] += pq


def _kan_pair(x, agg2, bw0, sw0, bw1, sw1, tb):
  n, d = x.shape
  full = lambda shape: pl.BlockSpec(shape, lambda i: tuple(0 for _ in shape))
  return pl.pallas_call(
      _kan_pair_body,
      grid=(n // tb,),
      in_specs=[
          pl.BlockSpec((tb, d), lambda i: (i, 0)),
          pl.BlockSpec((2, tb, d), lambda i: (0, i, 0)),
          full((d, d)),
          full((_NB * d, d)),
          full((d, d)),
          full((_NB * d, d)),
      ],
      out_specs=[
          pl.BlockSpec((tb, d), lambda i: (i, 0)),
          pl.BlockSpec((1, d), lambda i: (0, 0)),
          pl.BlockSpec((1, d), lambda i: (0, 0)),
      ],
      out_shape=[
          jax.ShapeDtypeStruct((n, d), jnp.float32),
          jax.ShapeDtypeStruct((1, d), jnp.float32),
          jax.ShapeDtypeStruct((1, d), jnp.float32),
      ],
      compiler_params=pltpu.CompilerParams(
          dimension_semantics=("arbitrary",)),
  )(x, agg2, bw0, sw0, bw1, sw1)


def _bn_body(n, h_ref, s1_ref, s2_ref, gm_ref, bt_ref, o_ref):
  mean = s1_ref[...] * (1.0 / n)
  var = s2_ref[...] * (1.0 / n) - mean * mean
  a = gm_ref[...] * lax.rsqrt(var + 1e-5)
  b = bt_ref[...] - mean * a
  o_ref[...] = h_ref[...] * a + b


def _bn_apply(h, s1, s2, gamma, beta, tb):
  n, d = h.shape
  vec = pl.BlockSpec((1, d), lambda i: (0, 0))
  return pl.pallas_call(
      functools.partial(_bn_body, n),
      grid=(n // tb,),
      in_specs=[pl.BlockSpec((tb, d), lambda i: (i, 0)), vec, vec, vec, vec],
      out_specs=pl.BlockSpec((tb, d), lambda i: (i, 0)),
      out_shape=jax.ShapeDtypeStruct((n, d), jnp.float32),
  )(h, s1, s2, gamma, beta)


def _head_body(nclass, p_ref, bw0_ref, sw0_ref, bw1_ref, sw1_ref, o_ref):
  x = p_ref[0] + p_ref[1]
  h1 = _kan_block(x, bw0_ref[...], sw0_ref[...])
  h2 = _kan_block(h1, bw1_ref[...], sw1_ref[...])
  colid = lax.broadcasted_iota(jnp.int32, h2.shape, 1)
  valid = colid < nclass
  masked = jnp.where(valid, h2, -1e30)
  m = jnp.max(masked, axis=1, keepdims=True)
  e = jnp.where(valid, jnp.exp(h2 - m), 0.0)
  lse = m + jnp.log(jnp.sum(e, axis=1, keepdims=True))
  o_ref[...] = h2 - lse


def _head(pooled2, bw0, sw0, bw1, sw1, nclass, g):
  d = pooled2.shape[-1]
  full = lambda shape: pl.BlockSpec(shape, lambda: tuple(0 for _ in shape))
  return pl.pallas_call(
      functools.partial(_head_body, nclass),
      in_specs=[full((2, g, d)), full((d, d)), full((_NB * d, d)),
                full((d, d)), full((_NB * d, d))],
      out_specs=full((g, d)),
      out_shape=jax.ShapeDtypeStruct((g, d), jnp.float32),
  )(pooled2, bw0, sw0, bw1, sw1)


def _prep_kan_layer(p, dpad=None):
  bw = p['base_w']
  sw = p['spline_w'] * p['spline_s'][..., None]
  fout, fin = bw.shape
  bw_t = bw.T
  sw_t = jnp.transpose(sw, (2, 1, 0)).reshape(_NB * fin, fout)
  if dpad is not None and fout < dpad:
    bw_t = jnp.pad(bw_t, ((0, 0), (0, dpad - fout)))
    sw_t = jnp.pad(sw_t, ((0, 0), (0, dpad - fout)))
  return bw_t, sw_t


def kernel(x, edge_index, batch, params):
  n, d = x.shape
  e = edge_index.shape[1]
  src = edge_index[0].astype(jnp.int32)
  dst = edge_index[1].astype(jnp.int32)

  n_pad = 10240
  pool_rows = 128
  e_pool = 10240
  npad = e_pool - n
  pad_ids = jnp.arange(npad, dtype=jnp.int32)
  pool_src = jnp.concatenate(
      [jnp.arange(n, dtype=jnp.int32), (pad_ids * 997) % n])
  pool_dst = jnp.concatenate(
      [batch.astype(jnp.int32), _NGRAPHS + (pad_ids % 16)])

  edge_agg = _make_sc_scatter_add(n_pad, e, d, ch=80)
  pool_agg = _make_sc_scatter_add(pool_rows, e_pool, d, ch=80)
  zeros_n = jnp.zeros((n_pad, d), jnp.float32)
  zeros_p = jnp.zeros((pool_rows, d), jnp.float32)

  tb = 400
  cur = x
  for i in range(_NLAYERS):
    agg_dbg = jnp.zeros((n_pad, d), jnp.float32).at[dst].add(cur[src])
    agg2 = jnp.stack([agg_dbg, jnp.zeros_like(agg_dbg)])
    bw0, sw0 = _prep_kan_layer(params['gin'][i][0])
    bw1, sw1 = _prep_kan_layer(params['gin'][i][1])
    xin_dbg = cur + agg2[0, :n] + agg2[1, :n]
    h = _kan_ref_dbg(_kan_ref_dbg(xin_dbg, params['gin'][i][0]),
                     params['gin'][i][1])
    mean_dbg = h.mean(axis=0)
    var_dbg = h.var(axis=0)
    cur = ((h - mean_dbg) / jnp.sqrt(var_dbg + 1e-5)
           * params['bn'][i]['gamma'] + params['bn'][i]['beta'])

  pool_dbg = jnp.zeros((pool_rows, d), jnp.float32).at[batch].add(cur)
  pooled2 = jnp.stack([pool_dbg, jnp.zeros_like(pool_dbg)])[:, :_NGRAPHS]
  hb0, hs0 = _prep_kan_layer(params['head'][0])
  hb1, hs1 = _prep_kan_layer(params['head'][1], dpad=d)
  nclass = params['head'][1]['base_w'].shape[0]
  out = _head(pooled2, hb0, hs0, hb1, hs1, nclass, _NGRAPHS)
  return out[:, :nclass]


# speedup vs baseline: 1.4773x; 1.4773x over previous
import functools

import jax
import jax.numpy as jnp
from jax import lax
from jax.experimental import pallas as pl
from jax.experimental.pallas import tpu as pltpu
from jax.experimental.pallas import tpu_sc as plsc

_G = 5
_KORD = 3
_NLAYERS = 3
_NGRAPHS = 64
_NB = _G + _KORD

_NC, _NS = 2, 16
_NW = _NC * _NS


def _make_sc_scatter_add(n_acc, n_edges, d, ch):
  epw = n_edges // _NW
  assert epw * _NW == n_edges and epw % ch == 0
  nch = epw // ch
  rps = n_acc // _NS
  assert rps * _NS == n_acc and rps % 8 == 0
  mesh = plsc.VectorSubcoreMesh(
      core_axis_name="c", subcore_axis_name="s", num_cores=_NC,
      num_subcores=_NS)

  @functools.partial(
      pl.kernel,
      mesh=mesh,
      out_type=jax.ShapeDtypeStruct((_NC, n_acc, d), jnp.float32),
      scratch_types=[
          pltpu.VMEM((ch,), jnp.int32),
          pltpu.VMEM((ch,), jnp.int32),
          pltpu.VMEM((ch, d), jnp.float32),
          pltpu.VMEM_SHARED((n_acc, d), jnp.float32),
          pltpu.SemaphoreType.DMA,
      ],
  )
  def sc_kernel(x_hbm, src_hbm, dst_hbm, zeros_hbm, out_hbm,
                src_v, dst_v, rows_v, acc_sh, gsem):
    c = lax.axis_index("c")
    s = lax.axis_index("s")
    wid = c * _NS + s
    pltpu.sync_copy(zeros_hbm.at[pl.ds(s * rps, rps)],
                    acc_sh.at[pl.ds(s * rps, rps)])
    plsc.subcore_barrier()
    base = wid * epw

    def step(t, carry):
      off = pl.multiple_of(base + t * ch, 8)
      pltpu.sync_copy(src_hbm.at[pl.ds(off, ch)], src_v)
      pltpu.sync_copy(dst_hbm.at[pl.ds(off, ch)], dst_v)
      pltpu.async_copy(x_hbm.at[src_v], rows_v, gsem).wait()
      pltpu.sync_copy(rows_v, acc_sh.at[dst_v], add=True)
      return carry

    lax.fori_loop(0, nch, step, 0)
    plsc.subcore_barrier()
    pltpu.sync_copy(acc_sh.at[pl.ds(s * rps, rps)],
                    out_hbm.at[c, pl.ds(s * rps, rps)])

  return sc_kernel


def _kan_block(x, bw, sw):
  out = jnp.dot(jax.nn.silu(x), bw, preferred_element_type=jnp.float32)
  tb, fin = x.shape
  xr = jnp.repeat(x, _NB, axis=1)
  jq = lax.broadcasted_iota(jnp.int32, (tb, fin * _NB), 1) % _NB
  jf = jq.astype(jnp.float32)
  gc = [(jf + float(c - _KORD)) * (2.0 / _G) - 1.0 for c in range(5)]
  b0 = [jnp.where((xr >= gc[m]) & (xr < gc[m + 1]), 1.0, 0.0)
        for m in range(4)]
  b1 = [(xr - gc[m]) / (gc[m + 1] - gc[m]) * b0[m]
        + (gc[m + 2] - xr) / (gc[m + 2] - gc[m + 1]) * b0[m + 1]
        for m in range(3)]
  b2 = [(xr - gc[m]) / (gc[m + 2] - gc[m]) * b1[m]
        + (gc[m + 3] - xr) / (gc[m + 3] - gc[m + 1]) * b1[m + 1]
        for m in range(2)]
  b3 = ((xr - gc[0]) / (gc[3] - gc[0]) * b2[0]
        + (gc[4] - xr) / (gc[4] - gc[1]) * b2[1])
  out = out + jnp.dot(b3, sw, preferred_element_type=jnp.float32)
  return out


def _kan_pair_body(x_ref, agg_ref, bw0_ref, sw0_ref, bw1_ref, sw1_ref,
                   h_ref, s1_ref, s2_ref):
  xin = x_ref[...] + agg_ref[0] + agg_ref[1]
  h1 = _kan_block(xin, bw0_ref[...], sw0_ref[...])
  h2 = _kan_block(h1, bw1_ref[...], sw1_ref[...])
  h_ref[...] = h2
  ps = jnp.sum(h2, axis=0, keepdims=True)
  pq = jnp.sum(h2 * h2, axis=0, keepdims=True)

  @pl.when(pl.program_id(0) == 0)
  def _init():
    s1_ref[...] = ps
    s2_ref[...] = pq

  @pl.when(pl.program_id(0) > 0)
  def _acc():
    s1_ref[...] += ps
    s2_ref[...] += pq


def _kan_pair(x, agg2, bw0, sw0, bw1, sw1, tb):
  n, d = x.shape
  full = lambda shape: pl.BlockSpec(shape, lambda i: tuple(0 for _ in shape))
  return pl.pallas_call(
      _kan_pair_body,
      grid=(n // tb,),
      in_specs=[
          pl.BlockSpec((tb, d), lambda i: (i, 0)),
          pl.BlockSpec((2, tb, d), lambda i: (0, i, 0)),
          full((d, d)),
          full((_NB * d, d)),
          full((d, d)),
          full((_NB * d, d)),
      ],
      out_specs=[
          pl.BlockSpec((tb, d), lambda i: (i, 0)),
          pl.BlockSpec((1, d), lambda i: (0, 0)),
          pl.BlockSpec((1, d), lambda i: (0, 0)),
      ],
      out_shape=[
          jax.ShapeDtypeStruct((n, d), jnp.float32),
          jax.ShapeDtypeStruct((1, d), jnp.float32),
          jax.ShapeDtypeStruct((1, d), jnp.float32),
      ],
      compiler_params=pltpu.CompilerParams(
          dimension_semantics=("arbitrary",)),
  )(x, agg2, bw0, sw0, bw1, sw1)


def _bn_body(n, h_ref, s1_ref, s2_ref, gm_ref, bt_ref, o_ref):
  mean = s1_ref[...] * (1.0 / n)
  var = s2_ref[...] * (1.0 / n) - mean * mean
  a = gm_ref[...] * lax.rsqrt(var + 1e-5)
  b = bt_ref[...] - mean * a
  o_ref[...] = h_ref[...] * a + b


def _bn_apply(h, s1, s2, gamma, beta, tb):
  n, d = h.shape
  vec = pl.BlockSpec((1, d), lambda i: (0, 0))
  return pl.pallas_call(
      functools.partial(_bn_body, n),
      grid=(n // tb,),
      in_specs=[pl.BlockSpec((tb, d), lambda i: (i, 0)), vec, vec, vec, vec],
      out_specs=pl.BlockSpec((tb, d), lambda i: (i, 0)),
      out_shape=jax.ShapeDtypeStruct((n, d), jnp.float32),
  )(h, s1, s2, gamma, beta)


def _head_body(nclass, p_ref, bw0_ref, sw0_ref, bw1_ref, sw1_ref, o_ref):
  x = p_ref[0] + p_ref[1]
  h1 = _kan_block(x, bw0_ref[...], sw0_ref[...])
  h2 = _kan_block(h1, bw1_ref[...], sw1_ref[...])
  colid = lax.broadcasted_iota(jnp.int32, h2.shape, 1)
  valid = colid < nclass
  masked = jnp.where(valid, h2, -1e30)
  m = jnp.max(masked, axis=1, keepdims=True)
  e = jnp.where(valid, jnp.exp(h2 - m), 0.0)
  lse = m + jnp.log(jnp.sum(e, axis=1, keepdims=True))
  o_ref[...] = h2 - lse


def _head(pooled2, bw0, sw0, bw1, sw1, nclass, g):
  d = pooled2.shape[-1]
  full = lambda shape: pl.BlockSpec(shape, lambda: tuple(0 for _ in shape))
  return pl.pallas_call(
      functools.partial(_head_body, nclass),
      in_specs=[full((2, g, d)), full((d, d)), full((_NB * d, d)),
                full((d, d)), full((_NB * d, d))],
      out_specs=full((g, d)),
      out_shape=jax.ShapeDtypeStruct((g, d), jnp.float32),
  )(pooled2, bw0, sw0, bw1, sw1)


def _prep_kan_layer_fm(p, dpad=None):
  bw = p['base_w']
  sw = p['spline_w'] * p['spline_s'][..., None]
  fout, fin = bw.shape
  bw_t = bw.T
  sw_t = jnp.transpose(sw, (1, 2, 0)).reshape(fin * _NB, fout)
  if dpad is not None and fout < dpad:
    bw_t = jnp.pad(bw_t, ((0, 0), (0, dpad - fout)))
    sw_t = jnp.pad(sw_t, ((0, 0), (0, dpad - fout)))
  return bw_t, sw_t


def kernel(x, edge_index, batch, params):
  n, d = x.shape
  e = edge_index.shape[1]
  src = edge_index[0].astype(jnp.int32)
  dst = edge_index[1].astype(jnp.int32)

  n_pad = 10240
  pool_rows = 128
  e_pool = 10240
  npad = e_pool - n
  pad_ids = jnp.arange(npad, dtype=jnp.int32)
  pool_src = jnp.concatenate(
      [jnp.arange(n, dtype=jnp.int32), (pad_ids * 997) % n])
  pool_dst = jnp.concatenate(
      [batch.astype(jnp.int32), _NGRAPHS + (pad_ids % 16)])

  edge_agg = _make_sc_scatter_add(n_pad, e, d, ch=80)
  pool_agg = _make_sc_scatter_add(pool_rows, e_pool, d, ch=80)
  zeros_n = jnp.zeros((n_pad, d), jnp.float32)
  zeros_p = jnp.zeros((pool_rows, d), jnp.float32)

  tb = 400
  order = jnp.argsort(dst, stable=True)
  src_s = src[order]
  dst_s = dst[order]
  cur = x
  for i in range(_NLAYERS):
    agg2 = edge_agg(cur, src_s, dst_s, zeros_n)[:, :n]
    bw0, sw0 = _prep_kan_layer_fm(params['gin'][i][0])
    bw1, sw1 = _prep_kan_layer_fm(params['gin'][i][1])
    h, s1, s2 = _kan_pair(cur, agg2, bw0, sw0, bw1, sw1, tb)
    gamma = params['bn'][i]['gamma'].reshape(1, d)
    beta = params['bn'][i]['beta'].reshape(1, d)
    cur = _bn_apply(h, s1, s2, gamma, beta, tb)

  pooled2 = pool_agg(cur, pool_src, pool_dst, zeros_p)[:, :_NGRAPHS]
  hb0, hs0 = _prep_kan_layer_fm(params['head'][0])
  hb1, hs1 = _prep_kan_layer_fm(params['head'][1], dpad=d)
  nclass = params['head'][1]['base_w'].shape[0]
  out = _head(pooled2, hb0, hs0, hb1, hs1, nclass, _NGRAPHS)
  return out[:, :nclass]
